# Initial kernel scaffold; baseline (speedup 1.0000x reference)
import jax
import jax.numpy as jnp
from jax.experimental import pallas as pl


def kernel(coords, features, train_coords, train_features, train_labels, W1, a1_src, a1_dst, b1, W2, a2_src, a2_dst, b2, W_fin, b_fin):
    raise NotImplementedError("write your pallas kernel here")


# speedup vs baseline: 1.2394x; 1.2394x over previous
import functools

import jax
import jax.numpy as jnp
from jax import lax
from jax.experimental import pallas as pl
from jax.experimental.pallas import tpu as pltpu
from jax.experimental.pallas import tpu_sc as plsc

N_TR = 20000
TPAD = 20096
B = 2048
K = 16
NN = K + 1
D_FEAT = 125
H = 128
TBL_D = 128
BQ = 128
NBLK = B // BQ


NCH = 4
CHR = TPAD // NCH


def _b16(x):
    return x.astype(jnp.bfloat16).astype(jnp.float32)


def _knn_body(tx_ref, ty_ref, cx_ref, cy_ref, out_ref, s_ref):
    cx = cx_ref[...]
    cy = cy_ref[...]
    cn = cx * cx + cy * cy
    cxb = _b16(cx)
    cyb = _b16(cy)
    inf = jnp.float32(jnp.inf)

    def init_chunk(c, carry):
        r = pl.ds(c * CHR, CHR)
        tx = tx_ref[r, :]
        ty = ty_ref[r, :]
        s_ref[r, :] = ((cn + (tx * tx + ty * ty))
                       - 2.0 * (_b16(tx) * cxb + _b16(ty) * cyb))
        return carry

    lax.fori_loop(0, NCH, init_chunk, 0)

    def one_iter(k, prev):
        m_prev, i_prev = prev

        def chunk(c, carry):
            m, i = carry
            r = pl.ds(c * CHR, CHR)
            rid = lax.broadcasted_iota(jnp.int32, (CHR, BQ), 0) + c * CHR
            sc = s_ref[r, :]
            valid = (sc > m_prev) | ((sc == m_prev) & (rid > i_prev))
            sv = jnp.where(valid, sc, inf)
            m_c = jnp.min(sv, axis=0, keepdims=True)
            i_c = jnp.min(jnp.where(valid & (sc == m_c), rid, TPAD),
                          axis=0, keepdims=True)
            i_new = jnp.where(m_c < m, i_c,
                              jnp.where(m_c == m, jnp.minimum(i, i_c), i))
            return jnp.minimum(m, m_c), i_new

        m_k, i_k = lax.fori_loop(
            0, NCH, chunk,
            (jnp.full((1, BQ), inf), jnp.full((1, BQ), TPAD, jnp.int32)))
        out_ref[pl.ds(k, 1), :] = i_k
        return m_k, i_k

    lax.fori_loop(0, K, one_iter,
                  (jnp.full((1, BQ), -jnp.float32(jnp.inf)),
                   jnp.full((1, BQ), -1, jnp.int32)))


def _knn(train_coords, coords):
    tc = jnp.pad(train_coords, ((0, TPAD - N_TR), (0, 0)), constant_values=1e6)
    tx = tc[:, 0:1]
    ty = tc[:, 1:2]
    cx = coords[:, 0].reshape(1, B)
    cy = coords[:, 1].reshape(1, B)
    return pl.pallas_call(
        _knn_body,
        grid=(NBLK,),
        in_specs=[
            pl.BlockSpec((TPAD, 1), lambda i: (0, 0)),
            pl.BlockSpec((TPAD, 1), lambda i: (0, 0)),
            pl.BlockSpec((1, BQ), lambda i: (0, i)),
            pl.BlockSpec((1, BQ), lambda i: (0, i)),
        ],
        out_specs=pl.BlockSpec((K, BQ), lambda i: (0, i)),
        out_shape=jax.ShapeDtypeStruct((K, B), jnp.int32),
        scratch_shapes=[pltpu.VMEM((TPAD, BQ), jnp.float32)],
        compiler_params=pltpu.CompilerParams(
            dimension_semantics=("arbitrary",)),
    )(tx, ty, cx, cy)


def _sc_gather(table, idx_flat):
    info = plsc.get_sparse_core_info()
    nw = info.num_cores * info.num_subcores
    n_rows = K * B
    bpw = n_rows // nw
    ch = 128
    mesh = plsc.VectorSubcoreMesh(core_axis_name="c", subcore_axis_name="s")

    @functools.partial(
        pl.kernel,
        mesh=mesh,
        out_type=jax.ShapeDtypeStruct((n_rows, TBL_D), jnp.float32),
        scratch_types=[
            pltpu.VMEM((ch,), jnp.int32),
            pltpu.VMEM((ch, TBL_D), jnp.float32),
            pltpu.SemaphoreType.DMA,
        ],
    )
    def gk(table_hbm, idx_hbm, out_hbm, idx_v, rows_v, sem):
        wid = lax.axis_index("s") * info.num_cores + lax.axis_index("c")
        base = wid * bpw
        for c in range(bpw // ch):
            off = base + c * ch
            pltpu.sync_copy(idx_hbm.at[pl.ds(off, ch)], idx_v)
            pltpu.async_copy(table_hbm.at[idx_v], rows_v, sem).wait()
            pltpu.sync_copy(rows_v, out_hbm.at[pl.ds(off, ch)])

    return gk(table, idx_flat)


def _gat_nodes(h_list, a_s, a_d, b, all_nodes):
    hb = [_b16(hj) for hj in h_list]
    a_sb = _b16(a_s)
    a_db = _b16(a_d)
    s = [jnp.sum(hj * a_sb, axis=1, keepdims=True) for hj in hb]
    d = [jnp.sum(hj * a_db, axis=1, keepdims=True) for hj in hb]
    s_row = jnp.concatenate(s, axis=1)
    outs = []
    for i in range(NN if all_nodes else 1):
        e = s_row + d[i]
        e = jnp.where(e >= 0, e, 0.2 * e)
        m = jnp.max(e, axis=1, keepdims=True)
        p = jnp.exp(e - m)
        z = jnp.sum(p, axis=1, keepdims=True)
        att = _b16(p / z)
        acc = att[:, 0:1] * hb[0]
        for j in range(1, NN):
            acc = acc + att[:, j : j + 1] * hb[j]
        outs.append(jnp.maximum(acc + b, 0.0))
    return outs


def _gat_body(g_ref, cin_ref, w1n_ref, w1c_ref, a1s_ref, a1d_ref, b1_ref,
              w2_ref, a2s_ref, a2d_ref, b2_ref, wf_ref, bf_ref, out_ref):
    f32 = jnp.float32
    w1cb = _b16(w1c_ref[...])
    w1nb = _b16(w1n_ref[...])
    h = [jnp.dot(_b16(cin_ref[...]), w1cb, preferred_element_type=f32)]
    for kk in range(K):
        h.append(jnp.dot(_b16(g_ref[kk]), w1nb, preferred_element_type=f32))
    x1 = _gat_nodes(h, a1s_ref[...], a1d_ref[...], b1_ref[...], True)
    w2b = _b16(w2_ref[...])
    h2 = [jnp.dot(_b16(x), w2b, preferred_element_type=f32) for x in x1]
    x2c = _gat_nodes(h2, a2s_ref[...], a2d_ref[...], b2_ref[...], False)[0]
    y = (jnp.dot(_b16(x2c), _b16(wf_ref[...]), preferred_element_type=f32)
         + bf_ref[...])
    out_ref[...] = jnp.maximum(y, 0.0)


def _gat(g, cin, w1n, w1c, a1s, a1d, b1, w2, a2s, a2d, b2, wf, bf):
    full = lambda shape: pl.BlockSpec(shape, lambda i: tuple(0 for _ in shape))
    return pl.pallas_call(
        _gat_body,
        grid=(NBLK,),
        in_specs=[
            pl.BlockSpec((K, BQ, TBL_D), lambda i: (0, i, 0)),
            pl.BlockSpec((BQ, TBL_D), lambda i: (i, 0)),
            full((TBL_D, H)),
            full((TBL_D, H)),
            full((1, H)),
            full((1, H)),
            full((1, H)),
            full((H, H)),
            full((1, H)),
            full((1, H)),
            full((1, H)),
            full((H, 1)),
            full((1, 1)),
        ],
        out_specs=pl.BlockSpec((BQ, 1), lambda i: (i, 0)),
        out_shape=jax.ShapeDtypeStruct((B, 1), jnp.float32),
        compiler_params=pltpu.CompilerParams(
            dimension_semantics=("arbitrary",)),
    )(g, cin, w1n, w1c, a1s, a1d, b1, w2, a2s, a2d, b2, wf, bf)


def kernel(coords, features, train_coords, train_features, train_labels,
           W1, a1_src, a1_dst, b1, W2, a2_src, a2_dst, b2, W_fin, b_fin):
    idx16 = _knn(train_coords, coords)
    table = jnp.concatenate(
        [train_features, train_labels, train_coords], axis=1)
    rows = _sc_gather(table, idx16.reshape(-1))
    g = rows.reshape(K, B, TBL_D)
    cin = jnp.concatenate(
        [features, coords, jnp.ones((B, 1), jnp.float32)], axis=1)
    w1n = jnp.concatenate([W1[2:127], W1[0:1], W1[127:129]], axis=0)
    w1c = jnp.concatenate([W1[2:127], W1[127:129], W1[1:2]], axis=0)
    r = lambda v: v.reshape(1, -1)
    return _gat(g, cin, w1n, w1c, r(a1_src), r(a1_dst), r(b1),
                W2, r(a2_src), r(a2_dst), r(b2), W_fin, r(b_fin))


# speedup vs baseline: 2.3930x; 1.9308x over previous
import functools

import jax
import jax.numpy as jnp
from jax import lax
from jax.experimental import pallas as pl
from jax.experimental.pallas import tpu as pltpu
from jax.experimental.pallas import tpu_sc as plsc

N_TR = 20000
TPAD = 20096
B = 2048
K = 16
NN = K + 1
D_FEAT = 125
H = 128
TBL_D = 128
BQ = 128
NBLK = B // BQ


NCH = 4
CHR = TPAD // NCH


def _b16(x):
    return x.astype(jnp.bfloat16).astype(jnp.float32)


SEG = 16
NS = TPAD // SEG
NSC = CHR // SEG


def _knn_body(tx_ref, ty_ref, cx_ref, cy_ref, out_ref, s_ref,
              rm0_ref, rm1_ref, rm2_ref, ri0_ref, ri1_ref, ri2_ref):
    cx = cx_ref[...]
    cy = cy_ref[...]
    cn = cx * cx + cy * cy
    cxb = _b16(cx)
    cyb = _b16(cy)
    inf = jnp.float32(jnp.inf)

    def init_chunk(c, carry):
        r = pl.ds(c * CHR, CHR)
        tx = tx_ref[r, :]
        ty = ty_ref[r, :]
        sc = ((cn + (tx * tx + ty * ty))
              - 2.0 * (_b16(tx) * cxb + _b16(ty) * cyb))
        s_ref[r, :] = sc
        v = sc.reshape(NSC, SEG, BQ)
        it = lax.broadcasted_iota(jnp.int32, (NSC, SEG, BQ), 1)
        m1 = jnp.min(v, axis=1)
        a1 = jnp.min(jnp.where(v == m1[:, None, :], it, SEG), axis=1)
        v2 = jnp.where(it == a1[:, None, :], inf, v)
        m2 = jnp.min(v2, axis=1)
        a2 = jnp.min(jnp.where(v2 == m2[:, None, :], it, SEG), axis=1)
        v3 = jnp.where(it == a2[:, None, :], inf, v2)
        m3 = jnp.min(v3, axis=1)
        a3 = jnp.min(jnp.where(v3 == m3[:, None, :], it, SEG), axis=1)
        base = (lax.broadcasted_iota(jnp.int32, (NSC, BQ), 0) * SEG + c * CHR)
        rs = pl.ds(c * NSC, NSC)
        rm0_ref[rs, :] = m1
        rm1_ref[rs, :] = m2
        rm2_ref[rs, :] = m3
        ri0_ref[rs, :] = base + a1
        ri1_ref[rs, :] = base + a2
        ri2_ref[rs, :] = base + a3
        return carry

    lax.fori_loop(0, NCH, init_chunk, 0)

    def full_scan(m_prev, i_prev):
        def chunk(c, carry):
            m, i = carry
            r = pl.ds(c * CHR, CHR)
            rid = lax.broadcasted_iota(jnp.int32, (CHR, BQ), 0) + c * CHR
            sc = s_ref[r, :]
            valid = (sc > m_prev) | ((sc == m_prev) & (rid > i_prev))
            sv = jnp.where(valid, sc, inf)
            m_c = jnp.min(sv, axis=0, keepdims=True)
            i_c = jnp.min(jnp.where(valid & (sc == m_c), rid, TPAD),
                          axis=0, keepdims=True)
            i_new = jnp.where(m_c < m, i_c,
                              jnp.where(m_c == m, jnp.minimum(i, i_c), i))
            return jnp.minimum(m, m_c), i_new

        return lax.fori_loop(
            0, NCH, chunk,
            (jnp.full((1, BQ), inf), jnp.full((1, BQ), TPAD, jnp.int32)))

    def one_iter(k, prev):
        m_prev, i_prev = prev
        rm0 = rm0_ref[...]
        rm1 = rm1_ref[...]
        rm2 = rm2_ref[...]
        ri0 = ri0_ref[...]
        ri1 = ri1_ref[...]
        ri2 = ri2_ref[...]
        v0 = (rm0 > m_prev) | ((rm0 == m_prev) & (ri0 > i_prev))
        v1 = (rm1 > m_prev) | ((rm1 == m_prev) & (ri1 > i_prev))
        v2 = (rm2 > m_prev) | ((rm2 == m_prev) & (ri2 > i_prev))
        val = jnp.where(v0, rm0, jnp.where(v1, rm1, jnp.where(v2, rm2, inf)))
        idx = jnp.where(v0, ri0, jnp.where(v1, ri1, jnp.where(v2, ri2, TPAD)))
        danger = jnp.any(jnp.logical_not(v0 | v1 | v2))
        m_r = jnp.min(val, axis=0, keepdims=True)
        i_r = jnp.min(jnp.where(val == m_r, idx, TPAD), axis=0, keepdims=True)
        m_k, i_k = lax.cond(danger,
                            lambda: full_scan(m_prev, i_prev),
                            lambda: (m_r, i_r))
        out_ref[pl.ds(k, 1), :] = i_k
        return m_k, i_k

    lax.fori_loop(0, K, one_iter,
                  (jnp.full((1, BQ), -jnp.float32(jnp.inf)),
                   jnp.full((1, BQ), -1, jnp.int32)))


def _knn(train_coords, coords):
    tc = jnp.pad(train_coords, ((0, TPAD - N_TR), (0, 0)), constant_values=1e6)
    tx = tc[:, 0:1]
    ty = tc[:, 1:2]
    cx = coords[:, 0].reshape(1, B)
    cy = coords[:, 1].reshape(1, B)
    return pl.pallas_call(
        _knn_body,
        grid=(NBLK,),
        in_specs=[
            pl.BlockSpec((TPAD, 1), lambda i: (0, 0)),
            pl.BlockSpec((TPAD, 1), lambda i: (0, 0)),
            pl.BlockSpec((1, BQ), lambda i: (0, i)),
            pl.BlockSpec((1, BQ), lambda i: (0, i)),
        ],
        out_specs=pl.BlockSpec((K, BQ), lambda i: (0, i)),
        out_shape=jax.ShapeDtypeStruct((K, B), jnp.int32),
        scratch_shapes=[
            pltpu.VMEM((TPAD, BQ), jnp.float32),
            pltpu.VMEM((NS, BQ), jnp.float32),
            pltpu.VMEM((NS, BQ), jnp.float32),
            pltpu.VMEM((NS, BQ), jnp.float32),
            pltpu.VMEM((NS, BQ), jnp.int32),
            pltpu.VMEM((NS, BQ), jnp.int32),
            pltpu.VMEM((NS, BQ), jnp.int32),
        ],
        compiler_params=pltpu.CompilerParams(
            dimension_semantics=("arbitrary",)),
    )(tx, ty, cx, cy)


def _sc_gather(table, idx_flat):
    info = plsc.get_sparse_core_info()
    nw = info.num_cores * info.num_subcores
    n_rows = K * B
    bpw = n_rows // nw
    ch = 128
    mesh = plsc.VectorSubcoreMesh(core_axis_name="c", subcore_axis_name="s")

    @functools.partial(
        pl.kernel,
        mesh=mesh,
        out_type=jax.ShapeDtypeStruct((n_rows, TBL_D), jnp.float32),
        scratch_types=[
            pltpu.VMEM((ch,), jnp.int32),
            pltpu.VMEM((ch, TBL_D), jnp.float32),
            pltpu.SemaphoreType.DMA,
        ],
    )
    def gk(table_hbm, idx_hbm, out_hbm, idx_v, rows_v, sem):
        wid = lax.axis_index("s") * info.num_cores + lax.axis_index("c")
        base = wid * bpw
        for c in range(bpw // ch):
            off = base + c * ch
            pltpu.sync_copy(idx_hbm.at[pl.ds(off, ch)], idx_v)
            pltpu.async_copy(table_hbm.at[idx_v], rows_v, sem).wait()
            pltpu.sync_copy(rows_v, out_hbm.at[pl.ds(off, ch)])

    return gk(table, idx_flat)


def _gat_nodes(h_list, a_s, a_d, b, all_nodes):
    hb = [_b16(hj) for hj in h_list]
    a_sb = _b16(a_s)
    a_db = _b16(a_d)
    s = [jnp.sum(hj * a_sb, axis=1, keepdims=True) for hj in hb]
    d = [jnp.sum(hj * a_db, axis=1, keepdims=True) for hj in hb]
    s_row = jnp.concatenate(s, axis=1)
    outs = []
    for i in range(NN if all_nodes else 1):
        e = s_row + d[i]
        e = jnp.where(e >= 0, e, 0.2 * e)
        m = jnp.max(e, axis=1, keepdims=True)
        p = jnp.exp(e - m)
        z = jnp.sum(p, axis=1, keepdims=True)
        att = _b16(p / z)
        acc = att[:, 0:1] * hb[0]
        for j in range(1, NN):
            acc = acc + att[:, j : j + 1] * hb[j]
        outs.append(jnp.maximum(acc + b, 0.0))
    return outs


def _gat_body(g_ref, cin_ref, w1n_ref, w1c_ref, a1s_ref, a1d_ref, b1_ref,
              w2_ref, a2s_ref, a2d_ref, b2_ref, wf_ref, bf_ref, out_ref):
    f32 = jnp.float32
    w1cb = _b16(w1c_ref[...])
    w1nb = _b16(w1n_ref[...])
    h = [jnp.dot(_b16(cin_ref[...]), w1cb, preferred_element_type=f32)]
    for kk in range(K):
        h.append(jnp.dot(_b16(g_ref[kk]), w1nb, preferred_element_type=f32))
    x1 = _gat_nodes(h, a1s_ref[...], a1d_ref[...], b1_ref[...], True)
    w2b = _b16(w2_ref[...])
    h2 = [jnp.dot(_b16(x), w2b, preferred_element_type=f32) for x in x1]
    x2c = _gat_nodes(h2, a2s_ref[...], a2d_ref[...], b2_ref[...], False)[0]
    y = (jnp.dot(_b16(x2c), _b16(wf_ref[...]), preferred_element_type=f32)
         + bf_ref[...])
    out_ref[...] = jnp.maximum(y, 0.0)


def _gat(g, cin, w1n, w1c, a1s, a1d, b1, w2, a2s, a2d, b2, wf, bf):
    full = lambda shape: pl.BlockSpec(shape, lambda i: tuple(0 for _ in shape))
    return pl.pallas_call(
        _gat_body,
        grid=(NBLK,),
        in_specs=[
            pl.BlockSpec((K, BQ, TBL_D), lambda i: (0, i, 0)),
            pl.BlockSpec((BQ, TBL_D), lambda i: (i, 0)),
            full((TBL_D, H)),
            full((TBL_D, H)),
            full((1, H)),
            full((1, H)),
            full((1, H)),
            full((H, H)),
            full((1, H)),
            full((1, H)),
            full((1, H)),
            full((H, 1)),
            full((1, 1)),
        ],
        out_specs=pl.BlockSpec((BQ, 1), lambda i: (i, 0)),
        out_shape=jax.ShapeDtypeStruct((B, 1), jnp.float32),
        compiler_params=pltpu.CompilerParams(
            dimension_semantics=("arbitrary",)),
    )(g, cin, w1n, w1c, a1s, a1d, b1, w2, a2s, a2d, b2, wf, bf)


def kernel(coords, features, train_coords, train_features, train_labels,
           W1, a1_src, a1_dst, b1, W2, a2_src, a2_dst, b2, W_fin, b_fin):
    idx16 = _knn(train_coords, coords)
    table = jnp.concatenate(
        [train_features, train_labels, train_coords], axis=1)
    rows = _sc_gather(table, idx16.reshape(-1))
    g = rows.reshape(K, B, TBL_D)
    cin = jnp.concatenate(
        [features, coords, jnp.ones((B, 1), jnp.float32)], axis=1)
    w1n = jnp.concatenate([W1[2:127], W1[0:1], W1[127:129]], axis=0)
    w1c = jnp.concatenate([W1[2:127], W1[127:129], W1[1:2]], axis=0)
    r = lambda v: v.reshape(1, -1)
    return _gat(g, cin, w1n, w1c, r(a1_src), r(a1_dst), r(b1),
                W2, r(a2_src), r(a2_dst), r(b2), W_fin, r(b_fin))


# speedup vs baseline: 3.2230x; 1.3469x over previous
import functools

import jax
import jax.numpy as jnp
from jax import lax
from jax.experimental import pallas as pl
from jax.experimental.pallas import tpu as pltpu
from jax.experimental.pallas import tpu_sc as plsc

N_TR = 20000
TPAD = 20224
B = 2048
K = 16
NN = K + 1
D_FEAT = 125
H = 128
TBL_D = 128
BQ = 128
NBLK = B // BQ


NCH = 4
CHR = TPAD // NCH


def _b16(x):
    return x.astype(jnp.bfloat16).astype(jnp.float32)


SEG = 64
NLEV = 4
NS = TPAD // SEG
NSC = CHR // SEG


def _knn_body(tx_ref, ty_ref, cx_ref, cy_ref, out_ref, *lev_refs):
    rm_refs = lev_refs[:NLEV]
    ri_refs = lev_refs[NLEV:]
    cx = cx_ref[...]
    cy = cy_ref[...]
    cn = cx * cx + cy * cy
    cxb = _b16(cx)
    cyb = _b16(cy)
    inf = jnp.float32(jnp.inf)

    def scores(c):
        tx = tx_ref[pl.ds(c * CHR, CHR), :]
        ty = ty_ref[pl.ds(c * CHR, CHR), :]
        return ((cn + (tx * tx + ty * ty))
                - 2.0 * (_b16(tx) * cxb + _b16(ty) * cyb))

    def init_chunk(c, carry):
        v = scores(c).reshape(NSC, SEG, BQ)
        it = lax.broadcasted_iota(jnp.int32, (NSC, SEG, BQ), 1)
        base = (lax.broadcasted_iota(jnp.int32, (NSC, BQ), 0) * SEG + c * CHR)
        rs = pl.ds(c * NSC, NSC)
        for lv in range(NLEV):
            m = jnp.min(v, axis=1)
            a = jnp.min(jnp.where(v == m[:, None, :], it, SEG), axis=1)
            rm_refs[lv][rs, :] = m
            ri_refs[lv][rs, :] = base + a
            if lv + 1 < NLEV:
                v = jnp.where(it == a[:, None, :], inf, v)
        return carry

    lax.fori_loop(0, NCH, init_chunk, 0)

    def full_scan(m_prev, i_prev):
        def chunk(c, carry):
            m, i = carry
            rid = lax.broadcasted_iota(jnp.int32, (CHR, BQ), 0) + c * CHR
            sc = scores(c)
            valid = (sc > m_prev) | ((sc == m_prev) & (rid > i_prev))
            sv = jnp.where(valid, sc, inf)
            m_c = jnp.min(sv, axis=0, keepdims=True)
            i_c = jnp.min(jnp.where(valid & (sc == m_c), rid, TPAD),
                          axis=0, keepdims=True)
            i_new = jnp.where(m_c < m, i_c,
                              jnp.where(m_c == m, jnp.minimum(i, i_c), i))
            return jnp.minimum(m, m_c), i_new

        return lax.fori_loop(
            0, NCH, chunk,
            (jnp.full((1, BQ), inf), jnp.full((1, BQ), TPAD, jnp.int32)))

    def one_iter(k, prev):
        m_prev, i_prev = prev
        val = jnp.full((NS, BQ), inf)
        idx = jnp.full((NS, BQ), TPAD, jnp.int32)
        alive = jnp.zeros((NS, BQ), jnp.bool_)
        for lv in range(NLEV - 1, -1, -1):
            rm = rm_refs[lv][...]
            ri = ri_refs[lv][...]
            v = (rm > m_prev) | ((rm == m_prev) & (ri > i_prev))
            val = jnp.where(v, rm, val)
            idx = jnp.where(v, ri, idx)
            alive = alive | v
        danger = jnp.any(jnp.logical_not(alive))
        m_r = jnp.min(val, axis=0, keepdims=True)
        i_r = jnp.min(jnp.where(val == m_r, idx, TPAD), axis=0, keepdims=True)
        m_k, i_k = lax.cond(danger,
                            lambda: full_scan(m_prev, i_prev),
                            lambda: (m_r, i_r))
        out_ref[pl.ds(k, 1), :] = i_k
        return m_k, i_k

    lax.fori_loop(0, K, one_iter,
                  (jnp.full((1, BQ), -jnp.float32(jnp.inf)),
                   jnp.full((1, BQ), -1, jnp.int32)))


def _knn(train_coords, coords):
    tc = jnp.pad(train_coords, ((0, TPAD - N_TR), (0, 0)), constant_values=1e6)
    tx = tc[:, 0:1]
    ty = tc[:, 1:2]
    cx = coords[:, 0].reshape(1, B)
    cy = coords[:, 1].reshape(1, B)
    return pl.pallas_call(
        _knn_body,
        grid=(NBLK,),
        in_specs=[
            pl.BlockSpec((TPAD, 1), lambda i: (0, 0)),
            pl.BlockSpec((TPAD, 1), lambda i: (0, 0)),
            pl.BlockSpec((1, BQ), lambda i: (0, i)),
            pl.BlockSpec((1, BQ), lambda i: (0, i)),
        ],
        out_specs=pl.BlockSpec((K, BQ), lambda i: (0, i)),
        out_shape=jax.ShapeDtypeStruct((K, B), jnp.int32),
        scratch_shapes=(
            [pltpu.VMEM((NS, BQ), jnp.float32) for _ in range(NLEV)]
            + [pltpu.VMEM((NS, BQ), jnp.int32) for _ in range(NLEV)]),
        compiler_params=pltpu.CompilerParams(
            dimension_semantics=("arbitrary",)),
    )(tx, ty, cx, cy)


def _sc_gather(table, idx_flat):
    info = plsc.get_sparse_core_info()
    nw = info.num_cores * info.num_subcores
    n_rows = K * B
    bpw = n_rows // nw
    ch = 128
    mesh = plsc.VectorSubcoreMesh(core_axis_name="c", subcore_axis_name="s")

    @functools.partial(
        pl.kernel,
        mesh=mesh,
        out_type=jax.ShapeDtypeStruct((n_rows, TBL_D), jnp.float32),
        scratch_types=[
            pltpu.VMEM((ch,), jnp.int32),
            pltpu.VMEM((ch, TBL_D), jnp.float32),
            pltpu.SemaphoreType.DMA,
        ],
    )
    def gk(table_hbm, idx_hbm, out_hbm, idx_v, rows_v, sem):
        wid = lax.axis_index("s") * info.num_cores + lax.axis_index("c")
        base = wid * bpw
        for c in range(bpw // ch):
            off = base + c * ch
            pltpu.sync_copy(idx_hbm.at[pl.ds(off, ch)], idx_v)
            pltpu.async_copy(table_hbm.at[idx_v], rows_v, sem).wait()
            pltpu.sync_copy(rows_v, out_hbm.at[pl.ds(off, ch)])

    return gk(table, idx_flat)


def _gat_nodes(h_list, a_s, a_d, b, all_nodes):
    hb = [_b16(hj) for hj in h_list]
    a_sb = _b16(a_s)
    a_db = _b16(a_d)
    s = [jnp.sum(hj * a_sb, axis=1, keepdims=True) for hj in hb]
    d = [jnp.sum(hj * a_db, axis=1, keepdims=True) for hj in hb]
    s_row = jnp.concatenate(s, axis=1)
    outs = []
    for i in range(NN if all_nodes else 1):
        e = s_row + d[i]
        e = jnp.where(e >= 0, e, 0.2 * e)
        m = jnp.max(e, axis=1, keepdims=True)
        p = jnp.exp(e - m)
        z = jnp.sum(p, axis=1, keepdims=True)
        att = _b16(p / z)
        acc = att[:, 0:1] * hb[0]
        for j in range(1, NN):
            acc = acc + att[:, j : j + 1] * hb[j]
        outs.append(jnp.maximum(acc + b, 0.0))
    return outs


def _gat_body(g_ref, cin_ref, w1n_ref, w1c_ref, a1s_ref, a1d_ref, b1_ref,
              w2_ref, a2s_ref, a2d_ref, b2_ref, wf_ref, bf_ref, out_ref):
    f32 = jnp.float32
    w1cb = _b16(w1c_ref[...])
    w1nb = _b16(w1n_ref[...])
    h = [jnp.dot(_b16(cin_ref[...]), w1cb, preferred_element_type=f32)]
    for kk in range(K):
        h.append(jnp.dot(_b16(g_ref[kk]), w1nb, preferred_element_type=f32))
    x1 = _gat_nodes(h, a1s_ref[...], a1d_ref[...], b1_ref[...], True)
    w2b = _b16(w2_ref[...])
    h2 = [jnp.dot(_b16(x), w2b, preferred_element_type=f32) for x in x1]
    x2c = _gat_nodes(h2, a2s_ref[...], a2d_ref[...], b2_ref[...], False)[0]
    y = (jnp.dot(_b16(x2c), _b16(wf_ref[...]), preferred_element_type=f32)
         + bf_ref[...])
    out_ref[...] = jnp.maximum(y, 0.0)


def _gat(g, cin, w1n, w1c, a1s, a1d, b1, w2, a2s, a2d, b2, wf, bf):
    full = lambda shape: pl.BlockSpec(shape, lambda i: tuple(0 for _ in shape))
    return pl.pallas_call(
        _gat_body,
        grid=(NBLK,),
        in_specs=[
            pl.BlockSpec((K, BQ, TBL_D), lambda i: (0, i, 0)),
            pl.BlockSpec((BQ, TBL_D), lambda i: (i, 0)),
            full((TBL_D, H)),
            full((TBL_D, H)),
            full((1, H)),
            full((1, H)),
            full((1, H)),
            full((H, H)),
            full((1, H)),
            full((1, H)),
            full((1, H)),
            full((H, 1)),
            full((1, 1)),
        ],
        out_specs=pl.BlockSpec((BQ, 1), lambda i: (i, 0)),
        out_shape=jax.ShapeDtypeStruct((B, 1), jnp.float32),
        compiler_params=pltpu.CompilerParams(
            dimension_semantics=("arbitrary",)),
    )(g, cin, w1n, w1c, a1s, a1d, b1, w2, a2s, a2d, b2, wf, bf)


def kernel(coords, features, train_coords, train_features, train_labels,
           W1, a1_src, a1_dst, b1, W2, a2_src, a2_dst, b2, W_fin, b_fin):
    idx16 = _knn(train_coords, coords)
    table = jnp.concatenate(
        [train_features, train_labels, train_coords], axis=1)
    rows = _sc_gather(table, idx16.reshape(-1))
    g = rows.reshape(K, B, TBL_D)
    cin = jnp.concatenate(
        [features, coords, jnp.ones((B, 1), jnp.float32)], axis=1)
    w1n = jnp.concatenate([W1[2:127], W1[0:1], W1[127:129]], axis=0)
    w1c = jnp.concatenate([W1[2:127], W1[127:129], W1[1:2]], axis=0)
    r = lambda v: v.reshape(1, -1)
    return _gat(g, cin, w1n, w1c, r(a1_src), r(a1_dst), r(b1),
                W2, r(a2_src), r(a2_dst), r(b2), W_fin, r(b_fin))


# speedup vs baseline: 3.8829x; 1.2047x over previous
import functools

import jax
import jax.numpy as jnp
from jax import lax
from jax.experimental import pallas as pl
from jax.experimental.pallas import tpu as pltpu
from jax.experimental.pallas import tpu_sc as plsc

N_TR = 20000
TPAD = 20224
B = 2048
K = 16
NN = K + 1
D_FEAT = 125
H = 128
TBL_D = 128
BQ = 128
NBLK = B // BQ


NCH = 2
CHR = TPAD // NCH


def _b16(x):
    return x.astype(jnp.bfloat16).astype(jnp.float32)


SEG = 128
NLEV = 4
NS = TPAD // SEG
NSC = CHR // SEG


def _knn_body(txy_ref, cxy_ref, out_ref, *lev_refs):
    rm_refs = lev_refs[:NLEV]
    ri_refs = lev_refs[NLEV:]
    cxy = cxy_ref[...]
    cx = cxy[0:1, :]
    cy = cxy[1:2, :]
    cn = cx * cx + cy * cy
    cxyb = _b16(cxy)
    inf = jnp.float32(jnp.inf)

    def scores(c):
        txy = txy_ref[pl.ds(c * CHR, CHR), :]
        tx = txy[:, 0:1]
        ty = txy[:, 1:2]
        dot = jnp.dot(_b16(txy), cxyb, preferred_element_type=jnp.float32)
        return (cn + (tx * tx + ty * ty)) - 2.0 * dot

    def init_chunk(c, carry):
        v = scores(c).reshape(NSC, SEG, BQ)
        it = lax.broadcasted_iota(jnp.int32, (NSC, SEG, BQ), 1)
        base = (lax.broadcasted_iota(jnp.int32, (NSC, BQ), 0) * SEG + c * CHR)
        rs = pl.ds(c * NSC, NSC)
        for lv in range(NLEV):
            m = jnp.min(v, axis=1)
            a = jnp.min(jnp.where(v == m[:, None, :], it, SEG), axis=1)
            rm_refs[lv][rs, :] = m
            ri_refs[lv][rs, :] = base + a
            if lv + 1 < NLEV:
                v = jnp.where(it == a[:, None, :], inf, v)
        return carry

    lax.fori_loop(0, NCH, init_chunk, 0)

    def full_scan(m_prev, i_prev):
        def chunk(c, carry):
            m, i = carry
            rid = lax.broadcasted_iota(jnp.int32, (CHR, BQ), 0) + c * CHR
            sc = scores(c)
            valid = (sc > m_prev) | ((sc == m_prev) & (rid > i_prev))
            sv = jnp.where(valid, sc, inf)
            m_c = jnp.min(sv, axis=0, keepdims=True)
            i_c = jnp.min(jnp.where(valid & (sc == m_c), rid, TPAD),
                          axis=0, keepdims=True)
            i_new = jnp.where(m_c < m, i_c,
                              jnp.where(m_c == m, jnp.minimum(i, i_c), i))
            return jnp.minimum(m, m_c), i_new

        return lax.fori_loop(
            0, NCH, chunk,
            (jnp.full((1, BQ), inf), jnp.full((1, BQ), TPAD, jnp.int32)))

    def one_iter(k, prev):
        m_prev, i_prev = prev
        val = jnp.full((NS, BQ), inf)
        idx = jnp.full((NS, BQ), TPAD, jnp.int32)
        alive = jnp.zeros((NS, BQ), jnp.bool_)
        for lv in range(NLEV - 1, -1, -1):
            rm = rm_refs[lv][...]
            ri = ri_refs[lv][...]
            v = (rm > m_prev) | ((rm == m_prev) & (ri > i_prev))
            val = jnp.where(v, rm, val)
            idx = jnp.where(v, ri, idx)
            alive = alive | v
        danger = jnp.any(jnp.logical_not(alive))
        m_r = jnp.min(val, axis=0, keepdims=True)
        i_r = jnp.min(jnp.where(val == m_r, idx, TPAD), axis=0, keepdims=True)
        m_k, i_k = lax.cond(danger,
                            lambda: full_scan(m_prev, i_prev),
                            lambda: (m_r, i_r))
        out_ref[pl.ds(k, 1), :] = i_k
        return m_k, i_k

    lax.fori_loop(0, K, one_iter,
                  (jnp.full((1, BQ), -jnp.float32(jnp.inf)),
                   jnp.full((1, BQ), -1, jnp.int32)))


def _knn(train_coords, coords):
    tc = jnp.pad(train_coords, ((0, TPAD - N_TR), (0, 0)), constant_values=1e6)
    cxy = coords.T
    return pl.pallas_call(
        _knn_body,
        grid=(NBLK,),
        in_specs=[
            pl.BlockSpec((TPAD, 2), lambda i: (0, 0)),
            pl.BlockSpec((2, BQ), lambda i: (0, i)),
        ],
        out_specs=pl.BlockSpec((K, BQ), lambda i: (0, i)),
        out_shape=jax.ShapeDtypeStruct((K, B), jnp.int32),
        scratch_shapes=(
            [pltpu.VMEM((NS, BQ), jnp.float32) for _ in range(NLEV)]
            + [pltpu.VMEM((NS, BQ), jnp.int32) for _ in range(NLEV)]),
        compiler_params=pltpu.CompilerParams(
            dimension_semantics=("arbitrary",)),
    )(tc, cxy)


def _sc_gather(table, idx_flat):
    info = plsc.get_sparse_core_info()
    nw = info.num_cores * info.num_subcores
    n_rows = K * B
    bpw = n_rows // nw
    ch = 128
    mesh = plsc.VectorSubcoreMesh(core_axis_name="c", subcore_axis_name="s")

    @functools.partial(
        pl.kernel,
        mesh=mesh,
        out_type=jax.ShapeDtypeStruct((n_rows, TBL_D), jnp.float32),
        scratch_types=[
            pltpu.VMEM((ch,), jnp.int32),
            pltpu.VMEM((ch, TBL_D), jnp.float32),
            pltpu.SemaphoreType.DMA,
        ],
    )
    def gk(table_hbm, idx_hbm, out_hbm, idx_v, rows_v, sem):
        wid = lax.axis_index("s") * info.num_cores + lax.axis_index("c")
        base = wid * bpw
        for c in range(bpw // ch):
            off = base + c * ch
            pltpu.sync_copy(idx_hbm.at[pl.ds(off, ch)], idx_v)
            pltpu.async_copy(table_hbm.at[idx_v], rows_v, sem).wait()
            pltpu.sync_copy(rows_v, out_hbm.at[pl.ds(off, ch)])

    return gk(table, idx_flat)


def _gat_nodes(h_list, a_s, a_d, b, all_nodes):
    hb = [_b16(hj) for hj in h_list]
    a_sb = _b16(a_s)
    a_db = _b16(a_d)
    s = [jnp.sum(hj * a_sb, axis=1, keepdims=True) for hj in hb]
    d = [jnp.sum(hj * a_db, axis=1, keepdims=True) for hj in hb]
    s_row = jnp.concatenate(s, axis=1)
    outs = []
    for i in range(NN if all_nodes else 1):
        e = s_row + d[i]
        e = jnp.where(e >= 0, e, 0.2 * e)
        m = jnp.max(e, axis=1, keepdims=True)
        p = jnp.exp(e - m)
        z = jnp.sum(p, axis=1, keepdims=True)
        att = _b16(p / z)
        acc = att[:, 0:1] * hb[0]
        for j in range(1, NN):
            acc = acc + att[:, j : j + 1] * hb[j]
        outs.append(jnp.maximum(acc + b, 0.0))
    return outs


def _gat_body(g_ref, cin_ref, w1n_ref, w1c_ref, a1s_ref, a1d_ref, b1_ref,
              w2_ref, a2s_ref, a2d_ref, b2_ref, wf_ref, bf_ref, out_ref):
    f32 = jnp.float32
    w1cb = _b16(w1c_ref[...])
    w1nb = _b16(w1n_ref[...])
    h = [jnp.dot(_b16(cin_ref[...]), w1cb, preferred_element_type=f32)]
    for kk in range(K):
        h.append(jnp.dot(_b16(g_ref[kk]), w1nb, preferred_element_type=f32))
    x1 = _gat_nodes(h, a1s_ref[...], a1d_ref[...], b1_ref[...], True)
    w2b = _b16(w2_ref[...])
    h2 = [jnp.dot(_b16(x), w2b, preferred_element_type=f32) for x in x1]
    x2c = _gat_nodes(h2, a2s_ref[...], a2d_ref[...], b2_ref[...], False)[0]
    y = (jnp.dot(_b16(x2c), _b16(wf_ref[...]), preferred_element_type=f32)
         + bf_ref[...])
    out_ref[...] = jnp.maximum(y, 0.0)


def _gat(g, cin, w1n, w1c, a1s, a1d, b1, w2, a2s, a2d, b2, wf, bf):
    full = lambda shape: pl.BlockSpec(shape, lambda i: tuple(0 for _ in shape))
    return pl.pallas_call(
        _gat_body,
        grid=(NBLK,),
        in_specs=[
            pl.BlockSpec((K, BQ, TBL_D), lambda i: (0, i, 0)),
            pl.BlockSpec((BQ, TBL_D), lambda i: (i, 0)),
            full((TBL_D, H)),
            full((TBL_D, H)),
            full((1, H)),
            full((1, H)),
            full((1, H)),
            full((H, H)),
            full((1, H)),
            full((1, H)),
            full((1, H)),
            full((H, 1)),
            full((1, 1)),
        ],
        out_specs=pl.BlockSpec((BQ, 1), lambda i: (i, 0)),
        out_shape=jax.ShapeDtypeStruct((B, 1), jnp.float32),
        compiler_params=pltpu.CompilerParams(
            dimension_semantics=("arbitrary",)),
    )(g, cin, w1n, w1c, a1s, a1d, b1, w2, a2s, a2d, b2, wf, bf)


def kernel(coords, features, train_coords, train_features, train_labels,
           W1, a1_src, a1_dst, b1, W2, a2_src, a2_dst, b2, W_fin, b_fin):
    idx16 = _knn(train_coords, coords)
    table = jnp.concatenate(
        [train_features, train_labels, train_coords], axis=1)
    rows = _sc_gather(table, idx16.reshape(-1))
    g = rows.reshape(K, B, TBL_D)
    cin = jnp.concatenate(
        [features, coords, jnp.ones((B, 1), jnp.float32)], axis=1)
    w1n = jnp.concatenate([W1[2:127], W1[0:1], W1[127:129]], axis=0)
    w1c = jnp.concatenate([W1[2:127], W1[127:129], W1[1:2]], axis=0)
    r = lambda v: v.reshape(1, -1)
    return _gat(g, cin, w1n, w1c, r(a1_src), r(a1_dst), r(b1),
                W2, r(a2_src), r(a2_dst), r(b2), W_fin, r(b_fin))


# speedup vs baseline: 3.8910x; 1.0021x over previous
import functools

import jax
import jax.numpy as jnp
from jax import lax
from jax.experimental import pallas as pl
from jax.experimental.pallas import tpu as pltpu
from jax.experimental.pallas import tpu_sc as plsc

N_TR = 20000
TPAD = 20224
B = 2048
K = 16
NN = K + 1
D_FEAT = 125
H = 128
TBL_D = 128
BQ = 128
NBLK = B // BQ


NCH = 2
CHR = TPAD // NCH


def _b16(x):
    return x.astype(jnp.bfloat16).astype(jnp.float32)


SEG = 128
NLEV = 4
NS = TPAD // SEG
NSC = CHR // SEG


def _knn_body(txy_ref, cxy_ref, out_ref, *lev_refs):
    rm_refs = lev_refs[:NLEV]
    ri_refs = lev_refs[NLEV:]
    cxy = cxy_ref[...]
    cx = cxy[0:1, :]
    cy = cxy[1:2, :]
    cn = cx * cx + cy * cy
    cxyb = _b16(cxy)
    inf = jnp.float32(jnp.inf)

    def scores(c):
        txy = txy_ref[pl.ds(c * CHR, CHR), :]
        tx = txy[:, 0:1]
        ty = txy[:, 1:2]
        dot = jnp.dot(_b16(txy), cxyb, preferred_element_type=jnp.float32)
        return (cn + (tx * tx + ty * ty)) - 2.0 * dot

    def init_chunk(c, carry):
        v = scores(c).reshape(NSC, SEG, BQ)
        it = lax.broadcasted_iota(jnp.int32, (NSC, SEG, BQ), 1)
        base = (lax.broadcasted_iota(jnp.int32, (NSC, BQ), 0) * SEG + c * CHR)
        rs = pl.ds(c * NSC, NSC)
        for lv in range(NLEV):
            m = jnp.min(v, axis=1)
            a = jnp.min(jnp.where(v == m[:, None, :], it, SEG), axis=1)
            rm_refs[lv][rs, :] = m
            ri_refs[lv][rs, :] = base + a
            if lv + 1 < NLEV:
                v = jnp.where(it == a[:, None, :], inf, v)
        return carry

    lax.fori_loop(0, NCH, init_chunk, 0)

    def full_scan(m_prev, i_prev):
        def chunk(c, carry):
            m, i = carry
            rid = lax.broadcasted_iota(jnp.int32, (CHR, BQ), 0) + c * CHR
            sc = scores(c)
            valid = (sc > m_prev) | ((sc == m_prev) & (rid > i_prev))
            sv = jnp.where(valid, sc, inf)
            m_c = jnp.min(sv, axis=0, keepdims=True)
            i_c = jnp.min(jnp.where(valid & (sc == m_c), rid, TPAD),
                          axis=0, keepdims=True)
            i_new = jnp.where(m_c < m, i_c,
                              jnp.where(m_c == m, jnp.minimum(i, i_c), i))
            return jnp.minimum(m, m_c), i_new

        return lax.fori_loop(
            0, NCH, chunk,
            (jnp.full((1, BQ), inf), jnp.full((1, BQ), TPAD, jnp.int32)))

    def one_iter(k, prev):
        m_prev, i_prev = prev
        val = jnp.full((NS, BQ), inf)
        idx = jnp.full((NS, BQ), TPAD, jnp.int32)
        alive = jnp.zeros((NS, BQ), jnp.bool_)
        for lv in range(NLEV - 1, -1, -1):
            rm = rm_refs[lv][...]
            ri = ri_refs[lv][...]
            v = (rm > m_prev) | ((rm == m_prev) & (ri > i_prev))
            val = jnp.where(v, rm, val)
            idx = jnp.where(v, ri, idx)
            alive = alive | v
        danger = jnp.any(jnp.logical_not(alive))
        m_r = jnp.min(val, axis=0, keepdims=True)
        i_r = jnp.min(jnp.where(val == m_r, idx, TPAD), axis=0, keepdims=True)
        m_k, i_k = lax.cond(danger,
                            lambda: full_scan(m_prev, i_prev),
                            lambda: (m_r, i_r))
        out_ref[pl.ds(k, 1), :] = i_k
        return m_k, i_k

    lax.fori_loop(0, K, one_iter,
                  (jnp.full((1, BQ), -jnp.float32(jnp.inf)),
                   jnp.full((1, BQ), -1, jnp.int32)))


def _knn(train_coords, coords):
    tc = jnp.pad(train_coords, ((0, TPAD - N_TR), (0, 0)), constant_values=1e6)
    cxy = coords.T
    return pl.pallas_call(
        _knn_body,
        grid=(NBLK,),
        in_specs=[
            pl.BlockSpec((TPAD, 2), lambda i: (0, 0)),
            pl.BlockSpec((2, BQ), lambda i: (0, i)),
        ],
        out_specs=pl.BlockSpec((K, BQ), lambda i: (0, i)),
        out_shape=jax.ShapeDtypeStruct((K, B), jnp.int32),
        scratch_shapes=(
            [pltpu.VMEM((NS, BQ), jnp.float32) for _ in range(NLEV)]
            + [pltpu.VMEM((NS, BQ), jnp.int32) for _ in range(NLEV)]),
        compiler_params=pltpu.CompilerParams(
            dimension_semantics=("arbitrary",)),
    )(tc, cxy)


def _sc_gather(table, idx_flat):
    info = plsc.get_sparse_core_info()
    nw = info.num_cores * info.num_subcores
    n_rows = K * B
    bpw = n_rows // nw
    ch = 128
    mesh = plsc.VectorSubcoreMesh(core_axis_name="c", subcore_axis_name="s")

    @functools.partial(
        pl.kernel,
        mesh=mesh,
        out_type=jax.ShapeDtypeStruct((n_rows, TBL_D), jnp.float32),
        scratch_types=[
            pltpu.VMEM((ch,), jnp.int32),
            pltpu.VMEM((ch, TBL_D), jnp.float32),
            pltpu.SemaphoreType.DMA,
        ],
    )
    def gk(table_hbm, idx_hbm, out_hbm, idx_v, rows_v, sem):
        wid = lax.axis_index("s") * info.num_cores + lax.axis_index("c")
        base = wid * bpw
        for c in range(bpw // ch):
            off = base + c * ch
            pltpu.sync_copy(idx_hbm.at[pl.ds(off, ch)], idx_v)
            pltpu.async_copy(table_hbm.at[idx_v], rows_v, sem).wait()
            pltpu.sync_copy(rows_v, out_hbm.at[pl.ds(off, ch)])

    return gk(table, idx_flat)


def _gat_nodes(h_all, a_s, a_d, b, all_nodes):
    hb_all = _b16(h_all)
    hb = [hb_all[j * BQ : (j + 1) * BQ] for j in range(NN)]
    s_all = jnp.sum(hb_all * _b16(a_s), axis=1, keepdims=True)
    d_all = jnp.sum(hb_all * _b16(a_d), axis=1, keepdims=True)
    s_row = jnp.concatenate(
        [s_all[j * BQ : (j + 1) * BQ] for j in range(NN)], axis=1)
    outs = []
    for i in range(NN if all_nodes else 1):
        e = s_row + d_all[i * BQ : (i + 1) * BQ]
        e = jnp.where(e >= 0, e, 0.2 * e)
        m = jnp.max(e, axis=1, keepdims=True)
        p = jnp.exp(e - m)
        z = jnp.sum(p, axis=1, keepdims=True)
        att = _b16(p / z)
        acc = att[:, 0:1] * hb[0]
        for j in range(1, NN):
            acc = acc + att[:, j : j + 1] * hb[j]
        outs.append(jnp.maximum(acc + b, 0.0))
    return outs


def _gat_body(g_ref, cin_ref, w1n_ref, w1c_ref, a1s_ref, a1d_ref, b1_ref,
              w2_ref, a2s_ref, a2d_ref, b2_ref, wf_ref, bf_ref, out_ref):
    f32 = jnp.float32
    hc = jnp.dot(_b16(cin_ref[...]), _b16(w1c_ref[...]),
                 preferred_element_type=f32)
    hn = jnp.dot(_b16(g_ref[...].reshape(K * BQ, TBL_D)), _b16(w1n_ref[...]),
                 preferred_element_type=f32)
    h_all = jnp.concatenate([hc, hn], axis=0)
    x1 = _gat_nodes(h_all, a1s_ref[...], a1d_ref[...], b1_ref[...], True)
    h2_all = jnp.dot(_b16(jnp.concatenate(x1, axis=0)), _b16(w2_ref[...]),
                     preferred_element_type=f32)
    x2c = _gat_nodes(h2_all, a2s_ref[...], a2d_ref[...], b2_ref[...], False)[0]
    y = (jnp.dot(_b16(x2c), _b16(wf_ref[...]), preferred_element_type=f32)
         + bf_ref[...])
    out_ref[...] = jnp.maximum(y, 0.0)


def _gat(g, cin, w1n, w1c, a1s, a1d, b1, w2, a2s, a2d, b2, wf, bf):
    full = lambda shape: pl.BlockSpec(shape, lambda i: tuple(0 for _ in shape))
    return pl.pallas_call(
        _gat_body,
        grid=(NBLK,),
        in_specs=[
            pl.BlockSpec((K, BQ, TBL_D), lambda i: (0, i, 0)),
            pl.BlockSpec((BQ, TBL_D), lambda i: (i, 0)),
            full((TBL_D, H)),
            full((TBL_D, H)),
            full((1, H)),
            full((1, H)),
            full((1, H)),
            full((H, H)),
            full((1, H)),
            full((1, H)),
            full((1, H)),
            full((H, 1)),
            full((1, 1)),
        ],
        out_specs=pl.BlockSpec((BQ, 1), lambda i: (i, 0)),
        out_shape=jax.ShapeDtypeStruct((B, 1), jnp.float32),
        compiler_params=pltpu.CompilerParams(
            dimension_semantics=("arbitrary",)),
    )(g, cin, w1n, w1c, a1s, a1d, b1, w2, a2s, a2d, b2, wf, bf)


def kernel(coords, features, train_coords, train_features, train_labels,
           W1, a1_src, a1_dst, b1, W2, a2_src, a2_dst, b2, W_fin, b_fin):
    idx16 = _knn(train_coords, coords)
    table = jnp.concatenate(
        [train_features, train_labels, train_coords], axis=1)
    rows = _sc_gather(table, idx16.reshape(-1))
    g = rows.reshape(K, B, TBL_D)
    cin = jnp.concatenate(
        [features, coords, jnp.ones((B, 1), jnp.float32)], axis=1)
    w1n = jnp.concatenate([W1[2:127], W1[0:1], W1[127:129]], axis=0)
    w1c = jnp.concatenate([W1[2:127], W1[127:129], W1[1:2]], axis=0)
    r = lambda v: v.reshape(1, -1)
    return _gat(g, cin, w1n, w1c, r(a1_src), r(a1_dst), r(b1),
                W2, r(a2_src), r(a2_dst), r(b2), W_fin, r(b_fin))


# speedup vs baseline: 3.9071x; 1.0041x over previous
import functools

import jax
import jax.numpy as jnp
from jax import lax
from jax.experimental import pallas as pl
from jax.experimental.pallas import tpu as pltpu
from jax.experimental.pallas import tpu_sc as plsc

N_TR = 20000
TPAD = 20224
B = 2048
K = 16
NN = K + 1
D_FEAT = 125
H = 128
TBL_D = 128
BQ = 128
NBLK = B // BQ


NCH = 2
CHR = TPAD // NCH


def _b16(x):
    return x.astype(jnp.bfloat16).astype(jnp.float32)


SEG = 128
NLEV = 4
NS = TPAD // SEG
NSC = CHR // SEG


def _knn_body(txy_ref, cxy_ref, out_ref, *lev_refs):
    rm_refs = lev_refs[:NLEV]
    ri_refs = lev_refs[NLEV:]
    cxy = cxy_ref[...]
    cx = cxy[0:1, :]
    cy = cxy[1:2, :]
    cn = cx * cx + cy * cy
    cxyb = _b16(cxy)
    inf = jnp.float32(jnp.inf)

    def scores(c):
        txy = txy_ref[pl.ds(c * CHR, CHR), :]
        tx = txy[:, 0:1]
        ty = txy[:, 1:2]
        dot = jnp.dot(_b16(txy), cxyb, preferred_element_type=jnp.float32)
        return (cn + (tx * tx + ty * ty)) - 2.0 * dot

    def init_chunk(c, carry):
        v = scores(c).reshape(NSC, SEG, BQ)
        it = lax.broadcasted_iota(jnp.int32, (NSC, SEG, BQ), 1)
        base = (lax.broadcasted_iota(jnp.int32, (NSC, BQ), 0) * SEG + c * CHR)
        rs = pl.ds(c * NSC, NSC)
        for lv in range(NLEV):
            m = jnp.min(v, axis=1)
            a = jnp.min(jnp.where(v == m[:, None, :], it, SEG), axis=1)
            rm_refs[lv][rs, :] = m
            ri_refs[lv][rs, :] = base + a
            if lv + 1 < NLEV:
                v = jnp.where(it == a[:, None, :], inf, v)
        return carry

    lax.fori_loop(0, NCH, init_chunk, 0)

    def full_scan(m_prev, i_prev):
        def chunk(c, carry):
            m, i = carry
            rid = lax.broadcasted_iota(jnp.int32, (CHR, BQ), 0) + c * CHR
            sc = scores(c)
            valid = (sc > m_prev) | ((sc == m_prev) & (rid > i_prev))
            sv = jnp.where(valid, sc, inf)
            m_c = jnp.min(sv, axis=0, keepdims=True)
            i_c = jnp.min(jnp.where(valid & (sc == m_c), rid, TPAD),
                          axis=0, keepdims=True)
            i_new = jnp.where(m_c < m, i_c,
                              jnp.where(m_c == m, jnp.minimum(i, i_c), i))
            return jnp.minimum(m, m_c), i_new

        return lax.fori_loop(
            0, NCH, chunk,
            (jnp.full((1, BQ), inf), jnp.full((1, BQ), TPAD, jnp.int32)))

    def one_iter(k, prev):
        m_prev, i_prev = prev
        val = jnp.full((NS, BQ), inf)
        idx = jnp.full((NS, BQ), TPAD, jnp.int32)
        alive = jnp.zeros((NS, BQ), jnp.bool_)
        for lv in range(NLEV - 1, -1, -1):
            rm = rm_refs[lv][...]
            ri = ri_refs[lv][...]
            v = (rm > m_prev) | ((rm == m_prev) & (ri > i_prev))
            val = jnp.where(v, rm, val)
            idx = jnp.where(v, ri, idx)
            alive = alive | v
        danger = jnp.any(jnp.logical_not(alive))
        m_r = jnp.min(val, axis=0, keepdims=True)
        i_r = jnp.min(jnp.where(val == m_r, idx, TPAD), axis=0, keepdims=True)
        m_k, i_k = lax.cond(danger,
                            lambda: full_scan(m_prev, i_prev),
                            lambda: (m_r, i_r))
        out_ref[pl.ds(k, 1), :] = i_k
        return m_k, i_k

    lax.fori_loop(0, K, one_iter,
                  (jnp.full((1, BQ), -jnp.float32(jnp.inf)),
                   jnp.full((1, BQ), -1, jnp.int32)))


def _knn(train_coords, coords):
    tc = jnp.pad(train_coords, ((0, TPAD - N_TR), (0, 0)), constant_values=1e6)
    cxy = coords.T
    return pl.pallas_call(
        _knn_body,
        grid=(NBLK,),
        in_specs=[
            pl.BlockSpec((TPAD, 2), lambda i: (0, 0)),
            pl.BlockSpec((2, BQ), lambda i: (0, i)),
        ],
        out_specs=pl.BlockSpec((K, BQ), lambda i: (0, i)),
        out_shape=jax.ShapeDtypeStruct((K, B), jnp.int32),
        scratch_shapes=(
            [pltpu.VMEM((NS, BQ), jnp.float32) for _ in range(NLEV)]
            + [pltpu.VMEM((NS, BQ), jnp.int32) for _ in range(NLEV)]),
        compiler_params=pltpu.CompilerParams(
            dimension_semantics=("arbitrary",)),
    )(tc, cxy)


def _sc_gather(table, idx_flat):
    info = plsc.get_sparse_core_info()
    nw = info.num_cores * info.num_subcores
    n_rows = K * B
    bpw = n_rows // nw
    ch = 128
    mesh = plsc.VectorSubcoreMesh(core_axis_name="c", subcore_axis_name="s")

    @functools.partial(
        pl.kernel,
        mesh=mesh,
        out_type=jax.ShapeDtypeStruct((n_rows, TBL_D), jnp.float32),
        scratch_types=[
            pltpu.VMEM((ch,), jnp.int32),
            pltpu.VMEM((ch, TBL_D), jnp.float32),
            pltpu.SemaphoreType.DMA,
        ],
    )
    def gk(table_hbm, idx_hbm, out_hbm, idx_v, rows_v, sem):
        wid = lax.axis_index("s") * info.num_cores + lax.axis_index("c")
        base = wid * bpw
        for c in range(bpw // ch):
            off = base + c * ch
            pltpu.sync_copy(idx_hbm.at[pl.ds(off, ch)], idx_v)
            pltpu.async_copy(table_hbm.at[idx_v], rows_v, sem).wait()
            pltpu.sync_copy(rows_v, out_hbm.at[pl.ds(off, ch)])

    return gk(table, idx_flat)


def _gat_nodes(h_all, a_s, a_d, b, all_nodes):
    hb_all = _b16(h_all)
    hb = [hb_all[j * BQ : (j + 1) * BQ] for j in range(NN)]
    s_all = jnp.sum(hb_all * _b16(a_s), axis=1, keepdims=True)
    d_all = jnp.sum(hb_all * _b16(a_d), axis=1, keepdims=True)
    s_row = jnp.concatenate(
        [s_all[j * BQ : (j + 1) * BQ] for j in range(NN)], axis=1)
    outs = []
    for i in range(NN if all_nodes else 1):
        e = s_row + d_all[i * BQ : (i + 1) * BQ]
        e = jnp.where(e >= 0, e, 0.2 * e)
        m = jnp.max(e, axis=1, keepdims=True)
        p = jnp.exp(e - m)
        z = jnp.sum(p, axis=1, keepdims=True)
        att = _b16(p / z)
        terms = [att[:, j : j + 1] * hb[j] for j in range(NN)]
        while len(terms) > 1:
            terms = [terms[t] + terms[t + 1] if t + 1 < len(terms)
                     else terms[t] for t in range(0, len(terms), 2)]
        outs.append(jnp.maximum(terms[0] + b, 0.0))
    return outs


def _gat_body(g_ref, cin_ref, w1n_ref, w1c_ref, a1s_ref, a1d_ref, b1_ref,
              w2_ref, a2s_ref, a2d_ref, b2_ref, wf_ref, bf_ref, out_ref):
    f32 = jnp.float32
    hc = jnp.dot(_b16(cin_ref[...]), _b16(w1c_ref[...]),
                 preferred_element_type=f32)
    hn = jnp.dot(_b16(g_ref[...].reshape(K * BQ, TBL_D)), _b16(w1n_ref[...]),
                 preferred_element_type=f32)
    h_all = jnp.concatenate([hc, hn], axis=0)
    x1 = _gat_nodes(h_all, a1s_ref[...], a1d_ref[...], b1_ref[...], True)
    h2_all = jnp.dot(_b16(jnp.concatenate(x1, axis=0)), _b16(w2_ref[...]),
                     preferred_element_type=f32)
    x2c = _gat_nodes(h2_all, a2s_ref[...], a2d_ref[...], b2_ref[...], False)[0]
    y = (jnp.dot(_b16(x2c), _b16(wf_ref[...]), preferred_element_type=f32)
         + bf_ref[...])
    out_ref[...] = jnp.maximum(y, 0.0)


def _gat(g, cin, w1n, w1c, a1s, a1d, b1, w2, a2s, a2d, b2, wf, bf):
    full = lambda shape: pl.BlockSpec(shape, lambda i: tuple(0 for _ in shape))
    return pl.pallas_call(
        _gat_body,
        grid=(NBLK,),
        in_specs=[
            pl.BlockSpec((K, BQ, TBL_D), lambda i: (0, i, 0)),
            pl.BlockSpec((BQ, TBL_D), lambda i: (i, 0)),
            full((TBL_D, H)),
            full((TBL_D, H)),
            full((1, H)),
            full((1, H)),
            full((1, H)),
            full((H, H)),
            full((1, H)),
            full((1, H)),
            full((1, H)),
            full((H, 1)),
            full((1, 1)),
        ],
        out_specs=pl.BlockSpec((BQ, 1), lambda i: (i, 0)),
        out_shape=jax.ShapeDtypeStruct((B, 1), jnp.float32),
        compiler_params=pltpu.CompilerParams(
            dimension_semantics=("arbitrary",)),
    )(g, cin, w1n, w1c, a1s, a1d, b1, w2, a2s, a2d, b2, wf, bf)


def kernel(coords, features, train_coords, train_features, train_labels,
           W1, a1_src, a1_dst, b1, W2, a2_src, a2_dst, b2, W_fin, b_fin):
    idx16 = _knn(train_coords, coords)
    table = jnp.concatenate(
        [train_features, train_labels, train_coords], axis=1)
    rows = _sc_gather(table, idx16.reshape(-1))
    g = rows.reshape(K, B, TBL_D)
    cin = jnp.concatenate(
        [features, coords, jnp.ones((B, 1), jnp.float32)], axis=1)
    w1n = jnp.concatenate([W1[2:127], W1[0:1], W1[127:129]], axis=0)
    w1c = jnp.concatenate([W1[2:127], W1[127:129], W1[1:2]], axis=0)
    r = lambda v: v.reshape(1, -1)
    return _gat(g, cin, w1n, w1c, r(a1_src), r(a1_dst), r(b1),
                W2, r(a2_src), r(a2_dst), r(b2), W_fin, r(b_fin))


# speedup vs baseline: 4.0916x; 1.0472x over previous
import functools

import jax
import jax.numpy as jnp
from jax import lax
from jax.experimental import pallas as pl
from jax.experimental.pallas import tpu as pltpu
from jax.experimental.pallas import tpu_sc as plsc

N_TR = 20000
TPAD = 20224
B = 2048
K = 16
NN = K + 1
D_FEAT = 125
H = 128
TBL_D = 128
BQ = 128
NBLK = B // BQ
BQG = 256
NBLKG = B // BQG


NCH = 2
CHR = TPAD // NCH


def _b16(x):
    return x.astype(jnp.bfloat16).astype(jnp.float32)


SEG = 128
NLEV = 4
NS = TPAD // SEG
NSC = CHR // SEG


def _knn_body(txy_ref, cxy_ref, out_ref, *lev_refs):
    rm_refs = lev_refs[:NLEV]
    ri_refs = lev_refs[NLEV:]
    cxy = cxy_ref[...]
    cx = cxy[0:1, :]
    cy = cxy[1:2, :]
    cn = cx * cx + cy * cy
    cxyb = _b16(cxy)
    inf = jnp.float32(jnp.inf)

    def scores(c):
        txy = txy_ref[pl.ds(c * CHR, CHR), :]
        tx = txy[:, 0:1]
        ty = txy[:, 1:2]
        dot = jnp.dot(_b16(txy), cxyb, preferred_element_type=jnp.float32)
        return (cn + (tx * tx + ty * ty)) - 2.0 * dot

    def init_chunk(c, carry):
        v = scores(c).reshape(NSC, SEG, BQ)
        it = lax.broadcasted_iota(jnp.int32, (NSC, SEG, BQ), 1)
        base = (lax.broadcasted_iota(jnp.int32, (NSC, BQ), 0) * SEG + c * CHR)
        rs = pl.ds(c * NSC, NSC)
        for lv in range(NLEV):
            m = jnp.min(v, axis=1)
            a = jnp.min(jnp.where(v == m[:, None, :], it, SEG), axis=1)
            rm_refs[lv][rs, :] = m
            ri_refs[lv][rs, :] = base + a
            if lv + 1 < NLEV:
                v = jnp.where(it == a[:, None, :], inf, v)
        return carry

    lax.fori_loop(0, NCH, init_chunk, 0)

    def full_scan(m_prev, i_prev):
        def chunk(c, carry):
            m, i = carry
            rid = lax.broadcasted_iota(jnp.int32, (CHR, BQ), 0) + c * CHR
            sc = scores(c)
            valid = (sc > m_prev) | ((sc == m_prev) & (rid > i_prev))
            sv = jnp.where(valid, sc, inf)
            m_c = jnp.min(sv, axis=0, keepdims=True)
            i_c = jnp.min(jnp.where(valid & (sc == m_c), rid, TPAD),
                          axis=0, keepdims=True)
            i_new = jnp.where(m_c < m, i_c,
                              jnp.where(m_c == m, jnp.minimum(i, i_c), i))
            return jnp.minimum(m, m_c), i_new

        return lax.fori_loop(
            0, NCH, chunk,
            (jnp.full((1, BQ), inf), jnp.full((1, BQ), TPAD, jnp.int32)))

    def one_iter(k, prev):
        m_prev, i_prev = prev
        val = jnp.full((NS, BQ), inf)
        idx = jnp.full((NS, BQ), TPAD, jnp.int32)
        alive = jnp.zeros((NS, BQ), jnp.bool_)
        for lv in range(NLEV - 1, -1, -1):
            rm = rm_refs[lv][...]
            ri = ri_refs[lv][...]
            v = (rm > m_prev) | ((rm == m_prev) & (ri > i_prev))
            val = jnp.where(v, rm, val)
            idx = jnp.where(v, ri, idx)
            alive = alive | v
        danger = jnp.any(jnp.logical_not(alive))
        m_r = jnp.min(val, axis=0, keepdims=True)
        i_r = jnp.min(jnp.where(val == m_r, idx, TPAD), axis=0, keepdims=True)
        m_k, i_k = lax.cond(danger,
                            lambda: full_scan(m_prev, i_prev),
                            lambda: (m_r, i_r))
        out_ref[pl.ds(k, 1), :] = i_k
        return m_k, i_k

    lax.fori_loop(0, K, one_iter,
                  (jnp.full((1, BQ), -jnp.float32(jnp.inf)),
                   jnp.full((1, BQ), -1, jnp.int32)))


def _knn(train_coords, coords):
    tc = jnp.pad(train_coords, ((0, TPAD - N_TR), (0, 0)), constant_values=1e6)
    cxy = coords.T
    return pl.pallas_call(
        _knn_body,
        grid=(NBLK,),
        in_specs=[
            pl.BlockSpec((TPAD, 2), lambda i: (0, 0)),
            pl.BlockSpec((2, BQ), lambda i: (0, i)),
        ],
        out_specs=pl.BlockSpec((K, BQ), lambda i: (0, i)),
        out_shape=jax.ShapeDtypeStruct((K, B), jnp.int32),
        scratch_shapes=(
            [pltpu.VMEM((NS, BQ), jnp.float32) for _ in range(NLEV)]
            + [pltpu.VMEM((NS, BQ), jnp.int32) for _ in range(NLEV)]),
        compiler_params=pltpu.CompilerParams(
            dimension_semantics=("arbitrary",)),
    )(tc, cxy)


def _sc_gather(table, idx_flat):
    info = plsc.get_sparse_core_info()
    nw = info.num_cores * info.num_subcores
    n_rows = K * B
    bpw = n_rows // nw
    ch = 128
    mesh = plsc.VectorSubcoreMesh(core_axis_name="c", subcore_axis_name="s")

    @functools.partial(
        pl.kernel,
        mesh=mesh,
        out_type=jax.ShapeDtypeStruct((n_rows, TBL_D), jnp.float32),
        scratch_types=[
            pltpu.VMEM((ch,), jnp.int32),
            pltpu.VMEM((ch, TBL_D), jnp.float32),
            pltpu.SemaphoreType.DMA,
        ],
    )
    def gk(table_hbm, idx_hbm, out_hbm, idx_v, rows_v, sem):
        wid = lax.axis_index("s") * info.num_cores + lax.axis_index("c")
        base = wid * bpw
        for c in range(bpw // ch):
            off = base + c * ch
            pltpu.sync_copy(idx_hbm.at[pl.ds(off, ch)], idx_v)
            pltpu.async_copy(table_hbm.at[idx_v], rows_v, sem).wait()
            pltpu.sync_copy(rows_v, out_hbm.at[pl.ds(off, ch)])

    return gk(table, idx_flat)


def _gat_nodes(h_all, a_s, a_d, b, all_nodes):
    hb_all = _b16(h_all)
    hb = [hb_all[j * BQG : (j + 1) * BQG] for j in range(NN)]
    s_all = jnp.sum(hb_all * _b16(a_s), axis=1, keepdims=True)
    d_all = jnp.sum(hb_all * _b16(a_d), axis=1, keepdims=True)
    s_row = jnp.concatenate(
        [s_all[j * BQG : (j + 1) * BQG] for j in range(NN)], axis=1)
    outs = []
    for i in range(NN if all_nodes else 1):
        e = s_row + d_all[i * BQG : (i + 1) * BQG]
        e = jnp.where(e >= 0, e, 0.2 * e)
        m = jnp.max(e, axis=1, keepdims=True)
        p = jnp.exp(e - m)
        z = jnp.sum(p, axis=1, keepdims=True)
        att = _b16(p / z)
        terms = [att[:, j : j + 1] * hb[j] for j in range(NN)]
        while len(terms) > 1:
            terms = [terms[t] + terms[t + 1] if t + 1 < len(terms)
                     else terms[t] for t in range(0, len(terms), 2)]
        outs.append(jnp.maximum(terms[0] + b, 0.0))
    return outs


def _gat_body(g_ref, cin_ref, w1n_ref, w1c_ref, a1s_ref, a1d_ref, b1_ref,
              w2_ref, a2s_ref, a2d_ref, b2_ref, wf_ref, bf_ref, out_ref):
    f32 = jnp.float32
    hc = jnp.dot(_b16(cin_ref[...]), _b16(w1c_ref[...]),
                 preferred_element_type=f32)
    hn = jnp.dot(_b16(g_ref[...].reshape(K * BQG, TBL_D)), _b16(w1n_ref[...]),
                 preferred_element_type=f32)
    h_all = jnp.concatenate([hc, hn], axis=0)
    x1 = _gat_nodes(h_all, a1s_ref[...], a1d_ref[...], b1_ref[...], True)
    h2_all = jnp.dot(_b16(jnp.concatenate(x1, axis=0)), _b16(w2_ref[...]),
                     preferred_element_type=f32)
    x2c = _gat_nodes(h2_all, a2s_ref[...], a2d_ref[...], b2_ref[...], False)[0]
    y = (jnp.dot(_b16(x2c), _b16(wf_ref[...]), preferred_element_type=f32)
         + bf_ref[...])
    out_ref[...] = jnp.maximum(y, 0.0)


def _gat(g, cin, w1n, w1c, a1s, a1d, b1, w2, a2s, a2d, b2, wf, bf):
    full = lambda shape: pl.BlockSpec(shape, lambda i: tuple(0 for _ in shape))
    return pl.pallas_call(
        _gat_body,
        grid=(NBLKG,),
        in_specs=[
            pl.BlockSpec((K, BQG, TBL_D), lambda i: (0, i, 0)),
            pl.BlockSpec((BQG, TBL_D), lambda i: (i, 0)),
            full((TBL_D, H)),
            full((TBL_D, H)),
            full((1, H)),
            full((1, H)),
            full((1, H)),
            full((H, H)),
            full((1, H)),
            full((1, H)),
            full((1, H)),
            full((H, 1)),
            full((1, 1)),
        ],
        out_specs=pl.BlockSpec((BQG, 1), lambda i: (i, 0)),
        out_shape=jax.ShapeDtypeStruct((B, 1), jnp.float32),
        compiler_params=pltpu.CompilerParams(
            dimension_semantics=("arbitrary",)),
    )(g, cin, w1n, w1c, a1s, a1d, b1, w2, a2s, a2d, b2, wf, bf)


def kernel(coords, features, train_coords, train_features, train_labels,
           W1, a1_src, a1_dst, b1, W2, a2_src, a2_dst, b2, W_fin, b_fin):
    idx16 = _knn(train_coords, coords)
    table = jnp.concatenate(
        [train_features, train_labels, train_coords], axis=1)
    rows = _sc_gather(table, idx16.reshape(-1))
    g = rows.reshape(K, B, TBL_D)
    cin = jnp.concatenate(
        [features, coords, jnp.ones((B, 1), jnp.float32)], axis=1)
    w1n = jnp.concatenate([W1[2:127], W1[0:1], W1[127:129]], axis=0)
    w1c = jnp.concatenate([W1[2:127], W1[127:129], W1[1:2]], axis=0)
    r = lambda v: v.reshape(1, -1)
    return _gat(g, cin, w1n, w1c, r(a1_src), r(a1_dst), r(b1),
                W2, r(a2_src), r(a2_dst), r(b2), W_fin, r(b_fin))
